# Initial kernel scaffold; baseline (speedup 1.0000x reference)
import jax
import jax.numpy as jnp
from jax.experimental import pallas as pl


def kernel(x, edge_index, edge_attr, phi0, psi0, phi1, psi1, phi2, psi2, Wr, br):
    raise NotImplementedError("write your pallas kernel here")


# speedup vs baseline: 2.9786x; 2.9786x over previous
import functools

import jax
import jax.numpy as jnp
from jax import lax
from jax.experimental import pallas as pl
from jax.experimental.pallas import tpu as pltpu
from jax.experimental.pallas import tpu_sc as plsc

N = 10000
NP = 10240
E = 160000
D = 256
HALF = 128
NSUB = 16
EPT = E // NSUB
CH = 128
NCHUNK = (EPT + CH - 1) // CH
EPT_PAD = NCHUNK * CH
RPT = NP // NSUB

_HIGH = lax.Precision.HIGHEST


def _mm(a, b):
    return lax.dot_general(a, b, (((1,), (0,)), ((), ())),
                           preferred_element_type=jnp.float32,
                           precision=_HIGH)


def _mmT(a, b):
    return lax.dot_general(a, b, (((1,), (1,)), ((), ())),
                           preferred_element_type=jnp.float32,
                           precision=_HIGH)


def _enorm_body(a_ref, o_ref):
    a = a_ref[...]
    sq = a * a
    lane = lax.broadcasted_iota(jnp.int32, (HALF, 8), 0)
    grp = lax.broadcasted_iota(jnp.int32, (HALF, 8), 1)
    sel = (lane // 16 == grp).astype(jnp.float32)
    o_ref[...] = jnp.sqrt(_mm(sq, sel))


EB = 2000


def _enorm(edge_attr):
    a = edge_attr.reshape(E // 8, HALF)
    out = pl.pallas_call(
        _enorm_body,
        grid=(E // 8 // EB,),
        in_specs=[pl.BlockSpec((EB, HALF), lambda i: (i, 0))],
        out_specs=pl.BlockSpec((EB, 8), lambda i: (i, 0)),
        out_shape=jax.ShapeDtypeStruct((E // 8, 8), jnp.float32),
    )(a)
    return out.reshape(E)


def _weights_body(phi0, psi0, phi1, psi1, phi2, psi2, wr, v_ref):
    b0 = _mm(psi1[...], psi0[...])
    b1 = _mm(psi1[...], phi0[...]) + _mm(phi1[...], psi0[...])
    b2 = _mm(phi1[...], phi0[...])
    c0 = _mm(psi2[...], b0)
    c1 = _mm(psi2[...], b1) + _mm(phi2[...], b0)
    c2 = _mm(psi2[...], b2) + _mm(phi2[...], b1)
    c3 = _mm(phi2[...], b2)
    w = wr[...]
    v_ref[...] = jnp.concatenate(
        [_mm(w, c0), _mm(w, c1), _mm(w, c2), _mm(w, c3)], axis=1)


def _weights(phi0, psi0, phi1, psi1, phi2, psi2, wr):
    return pl.pallas_call(
        _weights_body,
        out_shape=jax.ShapeDtypeStruct((D, 4 * D), jnp.float32),
    )(phi0, psi0, phi1, psi1, phi2, psi2, wr)


def _spmm_body(table, gidx_hbm, dst_hbm, en_hbm, y_out,
               gidx_v, dst_v, en_v, dstc_v, rows_v, acc, sem):
    c = lax.axis_index("c")
    s = lax.axis_index("s")
    cs = c * NSUB + s
    pltpu.sync_copy(gidx_hbm.at[cs], gidx_v)
    pltpu.sync_copy(dst_hbm.at[s], dst_v)
    pltpu.sync_copy(en_hbm.at[s], en_v)

    def zero_body(i, _):
        for k in range(8):
            rows_v[i, pl.ds(k * 16, 16)] = jnp.zeros((16,), jnp.float32)
        return 0

    lax.fori_loop(0, CH, zero_body, 0)
    base = s * RPT
    for q in range(5):
        ln = 128
        pltpu.sync_copy(rows_v.at[pl.ds(0, ln)],
                        acc.at[pl.ds(base + q * 128, ln)])
    plsc.subcore_barrier()

    def chunk_body(ch, _):
        pltpu.async_copy(table.at[gidx_v.at[ch]], rows_v, sem).wait()
        for k in range(CH // 16):
            dstc_v[pl.ds(k * 16, 16)] = dst_v[ch, pl.ds(k * 16, 16)]
        for g in range(CH // 16):
            ev = en_v[ch, pl.ds(g * 16, 16)]
            for j in range(16):
                r = g * 16 + j
                sc = ev[j]
                for k in range(HALF // 16):
                    sl = pl.ds(k * 16, 16)
                    rows_v[r, sl] = rows_v[r, sl] * sc
        pltpu.sync_copy(rows_v, acc.at[dstc_v], add=True)
        return 0

    lax.fori_loop(0, NCHUNK, chunk_body, 0)
    plsc.subcore_barrier()

    obase = c * NP + s * RPT
    for q in range(5):
        ln = 128
        pltpu.sync_copy(acc.at[pl.ds(base + q * 128, ln)],
                        rows_v.at[pl.ds(0, ln)])
        pltpu.sync_copy(rows_v.at[pl.ds(0, ln)],
                        y_out.at[pl.ds(obase + q * 128, ln)])


_spmm = pl.kernel(
    _spmm_body,
    out_type=jax.ShapeDtypeStruct((2 * NP, HALF), jnp.float32),
    mesh=plsc.VectorSubcoreMesh(core_axis_name="c", subcore_axis_name="s",
                                num_cores=2, num_subcores=NSUB),
    scratch_types=[
        pltpu.VMEM((NCHUNK, CH), jnp.int32),
        pltpu.VMEM((NCHUNK, CH), jnp.int32),
        pltpu.VMEM((NCHUNK, CH), jnp.float32),
        pltpu.VMEM((CH,), jnp.int32),
        pltpu.VMEM((CH, HALF), jnp.float32),
        pltpu.VMEM_SHARED((NP, HALF), jnp.float32),
        pltpu.SemaphoreType.DMA,
    ],
)


BN = 80


def _gemm_body(x, y1a, y1b, y2a, y2b, y3a, y3b, v, br, o_ref):
    vv = v[...]
    acc = _mmT(x[...], vv[:, 0:256])
    acc += _mmT(y1a[...], vv[:, 256:384])
    acc += _mmT(y1b[...], vv[:, 384:512])
    acc += _mmT(y2a[...], vv[:, 512:640])
    acc += _mmT(y2b[...], vv[:, 640:768])
    acc += _mmT(y3a[...], vv[:, 768:896])
    acc += _mmT(y3b[...], vv[:, 896:1024])
    o_ref[...] = acc + br[...]


def _gemm(x, y1, y2, y3, v, br):
    nblk = N // BN
    half_spec_a = pl.BlockSpec((BN, HALF), lambda i: (i, 0))
    half_spec_b = pl.BlockSpec((BN, HALF), lambda i: (i + NP // BN, 0))
    return pl.pallas_call(
        _gemm_body,
        grid=(nblk,),
        in_specs=[
            pl.BlockSpec((BN, D), lambda i: (i, 0)),
            half_spec_a, half_spec_b,
            half_spec_a, half_spec_b,
            half_spec_a, half_spec_b,
            pl.BlockSpec((D, 4 * D), lambda i: (0, 0)),
            pl.BlockSpec((1, D), lambda i: (0, 0)),
        ],
        out_specs=pl.BlockSpec((BN, D), lambda i: (i, 0)),
        out_shape=jax.ShapeDtypeStruct((N, D), jnp.float32),
    )(x, y1, y1, y2, y2, y3, y3, v, br)


def _pad_tiles(arr):
    a = arr.reshape(NSUB, EPT)
    a = jnp.pad(a, ((0, 0), (0, EPT_PAD - EPT)))
    return a.reshape(NSUB, NCHUNK, CH)


def kernel(x, edge_index, edge_attr, phi0, psi0, phi1, psi1, phi2, psi2,
           Wr, br):
    src = edge_index[0].astype(jnp.int32)
    dst = edge_index[1].astype(jnp.int32)

    enorm = _enorm(edge_attr)
    v = _weights(phi0, psi0, phi1, psi1, phi2, psi2, Wr)

    srcp = _pad_tiles(src)
    dstp = _pad_tiles(dst)
    enp = _pad_tiles(enorm)

    gidx1 = jnp.concatenate([2 * srcp[None], 2 * srcp[None] + 1], axis=0)
    gidx1 = gidx1.reshape(2 * NSUB, NCHUNK, CH)
    gidx2 = jnp.concatenate([srcp[None], srcp[None] + NP], axis=0)
    gidx2 = gidx2.reshape(2 * NSUB, NCHUNK, CH)

    x2 = x.reshape(2 * N, HALF)
    y1 = _spmm(x2, gidx1, dstp, enp)
    y2 = _spmm(y1, gidx2, dstp, enp)
    y3 = _spmm(y2, gidx2, dstp, enp)

    return _gemm(x, y1, y2, y3, v, br.reshape(1, D))
